# baseline (speedup 1.0000x reference)
import functools

import jax
import jax.numpy as jnp
from jax import lax
from jax.experimental import pallas as pl
from jax.experimental.pallas import tpu as pltpu
from jax.experimental.pallas import tpu_sc as plsc

N = 10000
E = 320000
D = 128
C = 3

NC = 2
NS = 16
NW = NC * NS
NPW = 320
N_LAST = N - (NW - 1) * NPW
B = 128
NB = E // B
LANES = D // 16


def _mm_body(x_ref, p_ref, wx_ref, wp_ref, wi_ref, bias_ref, g_ref, h_ref):
    xb = x_ref[...]
    pwp = jnp.dot(p_ref[...], wp_ref[...], preferred_element_type=jnp.float32)
    g_ref[...] = (jnp.dot(xb, wx_ref[...], preferred_element_type=jnp.float32)
                  + pwp)
    h_ref[...] = (jnp.dot(xb, wi_ref[...], preferred_element_type=jnp.float32)
                  - pwp + bias_ref[...])


def _offsets_body(nb_ref, off_ref):
    @pl.when(pl.program_id(0) == 0)
    def _():
        off_ref[...] = jnp.zeros((1, 128), jnp.int32)

    blk = nb_ref[...]
    row = off_ref[...]
    lane = lax.broadcasted_iota(jnp.int32, (1, 128), 1)
    for w in range(NW + 1):
        cnt = jnp.sum((blk < (w * NPW)).astype(jnp.int32))
        row = row + jnp.where(lane == w, cnt, 0)
    off_ref[...] = row


JUNK = NPW


def _segmax_body(g_hbm, h_hbm, src_hbm, dst_hbm, offs_hbm, out_hbm,
                 acc_v, h_v, rows0_v, rows1_v, idx0_v, idx1_v, dst0_v, dst1_v,
                 offs_v, m_v, p_v, gsem0, gsem1, isem0, isem1):
    cid = lax.axis_index("c")
    sid = lax.axis_index("s")
    wid = cid * NS + sid
    n_lo = wid * NPW

    pltpu.sync_copy(offs_hbm, offs_v)

    @pl.when(wid < NW - 1)
    def _():
        pltpu.sync_copy(h_hbm.at[pl.ds(n_lo, NPW)], h_v)

    @pl.when(wid == NW - 1)
    def _():
        pltpu.sync_copy(h_hbm.at[pl.ds(n_lo, N_LAST)], h_v.at[pl.ds(0, N_LAST)])

    ev = offs_v[pl.ds(wid, 16)]
    e_start = ev[0]
    e_end = ev[1]

    neg_inf = jnp.full((16,), -jnp.inf, jnp.float32)

    def init_row(r, carry):
        for j in range(LANES):
            acc_v[r, pl.ds(j * 16, 16)] = neg_inf
        return carry

    lax.fori_loop(0, NPW + 8, init_row, 0)
    for j in range(LANES):
        m_v[pl.ds(j * 16, 16)] = neg_inf
    p_v[pl.ds(0, 16)] = jnp.broadcast_to(jnp.int32(JUNK), (16,))

    b_first = e_start // B
    b_last = (e_end + B - 1) // B
    nb = b_last - b_first

    rows = (rows0_v, rows1_v)
    idxb = (idx0_v, idx1_v)
    dstb = (dst0_v, dst1_v)
    gsem = (gsem0, gsem1)
    isem = (isem0, isem1)

    def idx_start(rel, ph):
        base = pl.multiple_of((b_first + rel) * B, B)
        pltpu.make_async_copy(src_hbm.at[pl.ds(base, B)], idxb[ph], isem[ph]).start()
        pltpu.make_async_copy(dst_hbm.at[pl.ds(base, B)],
                              dstb[ph].at[pl.ds(0, B)], isem[ph]).start()

    def idx_wait(ph):
        pltpu.make_async_copy(src_hbm.at[pl.ds(0, B)], idxb[ph], isem[ph]).wait()
        pltpu.make_async_copy(dst_hbm.at[pl.ds(0, B)],
                              dstb[ph].at[pl.ds(0, B)], isem[ph]).wait()

    def gather_start(ph):
        pltpu.make_async_copy(g_hbm.at[idxb[ph]], rows[ph], gsem[ph]).start()

    def gather_wait(ph):
        pltpu.make_async_copy(g_hbm.at[idxb[ph]], rows[ph], gsem[ph]).wait()

    @pl.when(nb > 0)
    def _():
        idx_start(0, 0)
        idx_wait(0)
        gather_start(0)

    @pl.when(nb > 1)
    def _():
        idx_start(1, 1)

    lanev = lax.broadcasted_iota(jnp.int32, (16,), 0)

    def accumulate(rel, ph):
        base = (b_first + rel) * B
        i_lo = jnp.maximum(e_start - base, 0)
        i_hi = jnp.minimum(e_end - base, B)
        rbuf = rows[ph]
        dbuf = dstb[ph]

        def group_once(off):
            dv = dbuf[pl.ds(off, 16)] - n_lo
            gi = lanev + off
            valid = (gi >= i_lo) & (gi < i_hi)
            ldv = jnp.where(valid, dv, JUNK)
            prev_ld = p_v[pl.ds(0, 16)][0]
            u0 = ldv[0]
            same_seg = (u0 == ldv[15]) & (u0 != JUNK) & (u0 == prev_ld)

            @pl.when(same_seg)
            def _():
                for j in range(LANES):
                    sl = pl.ds(j * 16, 16)
                    v = [rbuf[off + k, sl] for k in range(16)]
                    while len(v) > 1:
                        v = [jnp.maximum(v[2 * t], v[2 * t + 1])
                             for t in range(len(v) // 2)]
                    m_v[sl] = jnp.maximum(m_v[sl], v[0])

            @pl.when(jnp.logical_not(same_seg))
            def _():
                m = [m_v[pl.ds(j * 16, 16)] for j in range(LANES)]
                prev = prev_ld
                for k in range(16):
                    ld = ldv[k]
                    rowk = [rbuf[off + k, pl.ds(j * 16, 16)]
                            for j in range(LANES)]
                    is_new = ld != prev

                    @pl.when(is_new)
                    def _(m=m, prev=prev):
                        for j in range(LANES):
                            sl = pl.ds(j * 16, 16)
                            acc_v[prev, sl] = jnp.maximum(
                                acc_v[prev, sl], m[j])

                    m = [jnp.where(is_new, rowk[j], jnp.maximum(m[j], rowk[j]))
                         for j in range(LANES)]
                    prev = ld
                for j in range(LANES):
                    m_v[pl.ds(j * 16, 16)] = m[j]

            p_v[pl.ds(0, 16)] = jnp.broadcast_to(ldv[15], (16,))

        def group_body(g, carry):
            off = pl.multiple_of(g * 32, 32)
            group_once(off)
            group_once(off + 16)
            return carry

        lax.fori_loop(0, B // 32, group_body, 0)

    def pair_body(p, carry):
        for ph in range(2):
            rel = 2 * p + ph

            @pl.when(rel < nb)
            def _(rel=rel, ph=ph):
                gather_wait(ph)

                @pl.when(rel + 1 < nb)
                def _():
                    idx_wait(1 - ph)
                    gather_start(1 - ph)

                accumulate(rel, ph)

                @pl.when(rel + 2 < nb)
                def _():
                    idx_start(rel + 2, ph)
        return carry

    lax.fori_loop(0, (nb + 1) // 2, pair_body, 0)

    last_ld = p_v[pl.ds(0, 16)][0]
    for j in range(LANES):
        sl = pl.ds(j * 16, 16)
        acc_v[last_ld, sl] = jnp.maximum(acc_v[last_ld, sl], m_v[sl])

    def fin_row(r, carry):
        for j in range(LANES):
            sl = pl.ds(j * 16, 16)
            acc_v[r, sl] = acc_v[r, sl] + h_v[r, sl]
        return carry

    lax.fori_loop(0, NPW, fin_row, 0)

    @pl.when(wid < NW - 1)
    def _():
        pltpu.sync_copy(acc_v.at[pl.ds(0, NPW)], out_hbm.at[pl.ds(n_lo, NPW)])

    @pl.when(wid == NW - 1)
    def _():
        pltpu.sync_copy(acc_v.at[pl.ds(0, N_LAST)],
                        out_hbm.at[pl.ds(n_lo, N_LAST)])


def kernel(pos, x, batch, neighbors, neighbors_batch, W_xi, b_xi, W_xn, b_xn):
    del batch
    src = neighbors.astype(jnp.int32)
    dst = neighbors_batch.astype(jnp.int32)

    wp = W_xn[:C]
    wx = W_xn[C:]
    bias = (b_xi + b_xn).reshape(1, D)

    bn = 1000
    g, h = pl.pallas_call(
        _mm_body,
        grid=(N // bn,),
        in_specs=[
            pl.BlockSpec((bn, D), lambda i: (i, 0)),
            pl.BlockSpec((bn, C), lambda i: (i, 0)),
            pl.BlockSpec((D, D), lambda i: (0, 0)),
            pl.BlockSpec((C, D), lambda i: (0, 0)),
            pl.BlockSpec((D, D), lambda i: (0, 0)),
            pl.BlockSpec((1, D), lambda i: (0, 0)),
        ],
        out_specs=[
            pl.BlockSpec((bn, D), lambda i: (i, 0)),
            pl.BlockSpec((bn, D), lambda i: (i, 0)),
        ],
        out_shape=[
            jax.ShapeDtypeStruct((N, D), jnp.float32),
            jax.ShapeDtypeStruct((N, D), jnp.float32),
        ],
    )(x, pos, wx, wp, W_xi, bias)

    nb2d = dst.reshape(NB, B)
    offs = pl.pallas_call(
        _offsets_body,
        grid=(1,),
        in_specs=[pl.BlockSpec((NB, B), lambda i: (0, 0))],
        out_specs=pl.BlockSpec((1, 128), lambda i: (0, 0)),
        out_shape=jax.ShapeDtypeStruct((1, 128), jnp.int32),
    )(nb2d).reshape(128)

    mesh = plsc.VectorSubcoreMesh(core_axis_name="c", subcore_axis_name="s",
                                  num_cores=NC, num_subcores=NS)
    out = pl.kernel(
        _segmax_body,
        out_type=jax.ShapeDtypeStruct((N, D), jnp.float32),
        mesh=mesh,
        scratch_types=[
            pltpu.VMEM((NPW + 8, D), jnp.float32),
            pltpu.VMEM((NPW, D), jnp.float32),
            pltpu.VMEM((B, D), jnp.float32),
            pltpu.VMEM((B, D), jnp.float32),
            pltpu.VMEM((B,), jnp.int32),
            pltpu.VMEM((B,), jnp.int32),
            pltpu.VMEM((B + 16,), jnp.int32),
            pltpu.VMEM((B + 16,), jnp.int32),
            pltpu.VMEM((128,), jnp.int32),
            pltpu.VMEM((D,), jnp.float32),
            pltpu.VMEM((16,), jnp.int32),
            pltpu.SemaphoreType.DMA,
            pltpu.SemaphoreType.DMA,
            pltpu.SemaphoreType.DMA,
            pltpu.SemaphoreType.DMA,
        ],
    )(g, h, src, dst, offs)

    return out


# speedup vs baseline: 1.4720x; 1.4720x over previous
import functools

import jax
import jax.numpy as jnp
from jax import lax
from jax.experimental import pallas as pl
from jax.experimental.pallas import tpu as pltpu
from jax.experimental.pallas import tpu_sc as plsc

N = 10000
E = 320000
D = 128
C = 3

NC = 2
NS = 16
NW = NC * NS
NPW = 320
N_LAST = N - (NW - 1) * NPW
B = 128
NB = E // B
LANES = D // 16


def _mm_body(x_ref, p_ref, wx_ref, wp_ref, wi_ref, bias_ref, g_ref, h_ref):
    xb = x_ref[...]
    pwp = jnp.dot(p_ref[...], wp_ref[...], preferred_element_type=jnp.float32)
    g_ref[...] = (jnp.dot(xb, wx_ref[...], preferred_element_type=jnp.float32)
                  + pwp)
    h_ref[...] = (jnp.dot(xb, wi_ref[...], preferred_element_type=jnp.float32)
                  - pwp + bias_ref[...])


def _offsets_body(nb_ref, off_ref):
    @pl.when(pl.program_id(0) == 0)
    def _():
        off_ref[...] = jnp.zeros((1, 128), jnp.int32)

    blk = nb_ref[...]
    row = off_ref[...]
    lane = lax.broadcasted_iota(jnp.int32, (1, 128), 1)
    for w in range(NW + 1):
        cnt = jnp.sum((blk < (w * NPW)).astype(jnp.int32))
        row = row + jnp.where(lane == w, cnt, 0)
    off_ref[...] = row


JUNK = NPW


def _segmax_body(g_hbm, h_hbm, src_hbm, dst_hbm, offs_hbm, out_hbm,
                 acc_v, h_v, rows0_v, rows1_v, idx0_v, idx1_v, dst0_v, dst1_v,
                 offs_v, m_v, p_v, gsem0, gsem1, isem0, isem1):
    cid = lax.axis_index("c")
    sid = lax.axis_index("s")
    wid = cid * NS + sid
    n_lo = wid * NPW

    pltpu.sync_copy(offs_hbm, offs_v)

    @pl.when(wid < NW - 1)
    def _():
        pltpu.sync_copy(h_hbm.at[pl.ds(n_lo, NPW)], h_v)

    @pl.when(wid == NW - 1)
    def _():
        pltpu.sync_copy(h_hbm.at[pl.ds(n_lo, N_LAST)], h_v.at[pl.ds(0, N_LAST)])

    ev = offs_v[pl.ds(wid, 16)]
    e_start = ev[0]
    e_end = ev[1]

    neg_inf = jnp.full((16,), -jnp.inf, jnp.float32)

    def init_row(r, carry):
        for j in range(LANES):
            acc_v[r, pl.ds(j * 16, 16)] = neg_inf
        return carry

    lax.fori_loop(0, NPW + 8, init_row, 0)
    for j in range(LANES):
        m_v[pl.ds(j * 16, 16)] = neg_inf
    p_v[pl.ds(0, 16)] = jnp.broadcast_to(jnp.int32(JUNK), (16,))

    b_first = e_start // B
    b_last = (e_end + B - 1) // B
    nb = b_last - b_first

    rows = (rows0_v, rows1_v)
    idxb = (idx0_v, idx1_v)
    dstb = (dst0_v, dst1_v)
    gsem = (gsem0, gsem1)
    isem = (isem0, isem1)

    def idx_start(rel, ph):
        base = pl.multiple_of((b_first + rel) * B, B)
        pltpu.make_async_copy(src_hbm.at[pl.ds(base, B)], idxb[ph], isem[ph]).start()
        pltpu.make_async_copy(dst_hbm.at[pl.ds(base, B)],
                              dstb[ph].at[pl.ds(0, B)], isem[ph]).start()

    def idx_wait(ph):
        pltpu.make_async_copy(src_hbm.at[pl.ds(0, B)], idxb[ph], isem[ph]).wait()
        pltpu.make_async_copy(dst_hbm.at[pl.ds(0, B)],
                              dstb[ph].at[pl.ds(0, B)], isem[ph]).wait()

    def gather_start(ph):
        pltpu.make_async_copy(g_hbm.at[idxb[ph]], rows[ph], gsem[ph]).start()

    def gather_wait(ph):
        pltpu.make_async_copy(g_hbm.at[idxb[ph]], rows[ph], gsem[ph]).wait()

    @pl.when(nb > 0)
    def _():
        idx_start(0, 0)
        idx_wait(0)
        gather_start(0)

    @pl.when(nb > 1)
    def _():
        idx_start(1, 1)

    lanev = lax.broadcasted_iota(jnp.int32, (16,), 0)

    def accumulate(rel, ph):
        base = (b_first + rel) * B
        i_lo = jnp.maximum(e_start - base, 0)
        i_hi = jnp.minimum(e_end - base, B)
        rbuf = rows[ph]
        dbuf = dstb[ph]

        def group_body(g, carry):
            off = g * 16
            dv = dbuf[pl.ds(off, 16)] - n_lo
            gi = lanev + off
            valid = (gi >= i_lo) & (gi < i_hi)
            ldv = jnp.where(valid, dv, JUNK)
            prev_ld = p_v[pl.ds(0, 16)][0]
            u0 = ldv[0]
            same_seg = (u0 == ldv[15]) & (u0 != JUNK) & (u0 == prev_ld)

            @pl.when(same_seg)
            def _():
                for j in range(LANES):
                    sl = pl.ds(j * 16, 16)
                    v = [rbuf[off + k, sl] for k in range(16)]
                    while len(v) > 1:
                        v = [jnp.maximum(v[2 * t], v[2 * t + 1])
                             for t in range(len(v) // 2)]
                    m_v[sl] = jnp.maximum(m_v[sl], v[0])

            @pl.when(jnp.logical_not(same_seg))
            def _():
                m = [m_v[pl.ds(j * 16, 16)] for j in range(LANES)]
                prev = prev_ld
                for k in range(16):
                    ld = ldv[k]
                    rowk = [rbuf[off + k, pl.ds(j * 16, 16)]
                            for j in range(LANES)]
                    is_new = ld != prev

                    @pl.when(is_new)
                    def _(m=m, prev=prev):
                        for j in range(LANES):
                            sl = pl.ds(j * 16, 16)
                            acc_v[prev, sl] = jnp.maximum(
                                acc_v[prev, sl], m[j])

                    m = [jnp.where(is_new, rowk[j], jnp.maximum(m[j], rowk[j]))
                         for j in range(LANES)]
                    prev = ld
                for j in range(LANES):
                    m_v[pl.ds(j * 16, 16)] = m[j]

            p_v[pl.ds(0, 16)] = jnp.broadcast_to(ldv[15], (16,))
            return carry

        lax.fori_loop(0, B // 16, group_body, 0)

    def pair_body(p, carry):
        for ph in range(2):
            rel = 2 * p + ph

            @pl.when(rel < nb)
            def _(rel=rel, ph=ph):
                gather_wait(ph)

                @pl.when(rel + 1 < nb)
                def _():
                    idx_wait(1 - ph)
                    gather_start(1 - ph)

                accumulate(rel, ph)

                @pl.when(rel + 2 < nb)
                def _():
                    idx_start(rel + 2, ph)
        return carry

    lax.fori_loop(0, (nb + 1) // 2, pair_body, 0)

    last_ld = p_v[pl.ds(0, 16)][0]
    for j in range(LANES):
        sl = pl.ds(j * 16, 16)
        acc_v[last_ld, sl] = jnp.maximum(acc_v[last_ld, sl], m_v[sl])

    def fin_row(r, carry):
        for j in range(LANES):
            sl = pl.ds(j * 16, 16)
            acc_v[r, sl] = acc_v[r, sl] + h_v[r, sl]
        return carry

    lax.fori_loop(0, NPW, fin_row, 0)

    @pl.when(wid < NW - 1)
    def _():
        pltpu.sync_copy(acc_v.at[pl.ds(0, NPW)], out_hbm.at[pl.ds(n_lo, NPW)])

    @pl.when(wid == NW - 1)
    def _():
        pltpu.sync_copy(acc_v.at[pl.ds(0, N_LAST)],
                        out_hbm.at[pl.ds(n_lo, N_LAST)])


def kernel(pos, x, batch, neighbors, neighbors_batch, W_xi, b_xi, W_xn, b_xn):
    del batch
    src = neighbors.astype(jnp.int32)
    dst = neighbors_batch.astype(jnp.int32)

    wp = W_xn[:C]
    wx = W_xn[C:]
    bias = (b_xi + b_xn).reshape(1, D)

    bn = 1000
    g, h = pl.pallas_call(
        _mm_body,
        grid=(N // bn,),
        in_specs=[
            pl.BlockSpec((bn, D), lambda i: (i, 0)),
            pl.BlockSpec((bn, C), lambda i: (i, 0)),
            pl.BlockSpec((D, D), lambda i: (0, 0)),
            pl.BlockSpec((C, D), lambda i: (0, 0)),
            pl.BlockSpec((D, D), lambda i: (0, 0)),
            pl.BlockSpec((1, D), lambda i: (0, 0)),
        ],
        out_specs=[
            pl.BlockSpec((bn, D), lambda i: (i, 0)),
            pl.BlockSpec((bn, D), lambda i: (i, 0)),
        ],
        out_shape=[
            jax.ShapeDtypeStruct((N, D), jnp.float32),
            jax.ShapeDtypeStruct((N, D), jnp.float32),
        ],
    )(x, pos, wx, wp, W_xi, bias)

    nb2d = dst.reshape(NB, B)
    offs = pl.pallas_call(
        _offsets_body,
        grid=(1,),
        in_specs=[pl.BlockSpec((NB, B), lambda i: (0, 0))],
        out_specs=pl.BlockSpec((1, 128), lambda i: (0, 0)),
        out_shape=jax.ShapeDtypeStruct((1, 128), jnp.int32),
    )(nb2d).reshape(128)

    mesh = plsc.VectorSubcoreMesh(core_axis_name="c", subcore_axis_name="s",
                                  num_cores=NC, num_subcores=NS)
    out = pl.kernel(
        _segmax_body,
        out_type=jax.ShapeDtypeStruct((N, D), jnp.float32),
        mesh=mesh,
        scratch_types=[
            pltpu.VMEM((NPW + 8, D), jnp.float32),
            pltpu.VMEM((NPW, D), jnp.float32),
            pltpu.VMEM((B, D), jnp.float32),
            pltpu.VMEM((B, D), jnp.float32),
            pltpu.VMEM((B,), jnp.int32),
            pltpu.VMEM((B,), jnp.int32),
            pltpu.VMEM((B + 16,), jnp.int32),
            pltpu.VMEM((B + 16,), jnp.int32),
            pltpu.VMEM((128,), jnp.int32),
            pltpu.VMEM((D,), jnp.float32),
            pltpu.VMEM((16,), jnp.int32),
            pltpu.SemaphoreType.DMA,
            pltpu.SemaphoreType.DMA,
            pltpu.SemaphoreType.DMA,
            pltpu.SemaphoreType.DMA,
        ],
    )(g, h, src, dst, offs)

    return out


# speedup vs baseline: 1.6114x; 1.0947x over previous
import functools

import jax
import jax.numpy as jnp
from jax import lax
from jax.experimental import pallas as pl
from jax.experimental.pallas import tpu as pltpu
from jax.experimental.pallas import tpu_sc as plsc

N = 10000
E = 320000
D = 128
C = 3

NC = 2
NS = 16
NW = NC * NS
NPW = 320
N_LAST = N - (NW - 1) * NPW
B = 128
NB = E // B
LANES = D // 16


def _mm_body(x_ref, p_ref, wx_ref, wp_ref, wi_ref, bias_ref, g_ref, h_ref):
    xb = x_ref[...]
    pwp = jnp.dot(p_ref[...], wp_ref[...], preferred_element_type=jnp.float32)
    g_ref[...] = (jnp.dot(xb, wx_ref[...], preferred_element_type=jnp.float32)
                  + pwp)
    h_ref[...] = (jnp.dot(xb, wi_ref[...], preferred_element_type=jnp.float32)
                  - pwp + bias_ref[...])


def _offsets_body(nb_ref, off_ref):
    @pl.when(pl.program_id(0) == 0)
    def _():
        off_ref[...] = jnp.zeros((1, 128), jnp.int32)

    blk = nb_ref[...]
    row = off_ref[...]
    lane = lax.broadcasted_iota(jnp.int32, (1, 128), 1)
    for w in range(NW + 1):
        cnt = jnp.sum((blk < (w * NPW)).astype(jnp.int32))
        row = row + jnp.where(lane == w, cnt, 0)
    off_ref[...] = row


JUNK = NPW


def _segmax_body(g_hbm, h_hbm, src_hbm, dst_hbm, offs_hbm, out_hbm,
                 acc_v, h_v, rows0_v, rows1_v, idx0_v, idx1_v, dst0_v, dst1_v,
                 offs_v, m_v, p_v, gsem0, gsem1, isem0, isem1):
    cid = lax.axis_index("c")
    sid = lax.axis_index("s")
    wid = cid * NS + sid
    n_lo = wid * NPW

    pltpu.sync_copy(offs_hbm, offs_v)

    @pl.when(wid < NW - 1)
    def _():
        pltpu.sync_copy(h_hbm.at[pl.ds(n_lo, NPW)], h_v)

    @pl.when(wid == NW - 1)
    def _():
        pltpu.sync_copy(h_hbm.at[pl.ds(n_lo, N_LAST)], h_v.at[pl.ds(0, N_LAST)])

    ev = offs_v[pl.ds(wid, 16)]
    e_start = ev[0]
    e_end = ev[1]

    neg_inf = jnp.full((16,), -jnp.inf, jnp.float32)

    def init_row(r, carry):
        for j in range(LANES):
            acc_v[r, pl.ds(j * 16, 16)] = neg_inf
        return carry

    lax.fori_loop(0, NPW + 8, init_row, 0)
    for j in range(LANES):
        m_v[pl.ds(j * 16, 16)] = neg_inf
    p_v[pl.ds(0, 16)] = jnp.broadcast_to(jnp.int32(JUNK), (16,))

    b_first = e_start // B
    b_last = (e_end + B - 1) // B
    nb = b_last - b_first

    rows = (rows0_v, rows1_v)
    idxb = (idx0_v, idx1_v)
    dstb = (dst0_v, dst1_v)
    gsem = (gsem0, gsem1)
    isem = (isem0, isem1)

    def idx_start(rel, ph):
        base = pl.multiple_of((b_first + rel) * B, B)
        pltpu.make_async_copy(src_hbm.at[pl.ds(base, B)], idxb[ph], isem[ph]).start()
        pltpu.make_async_copy(dst_hbm.at[pl.ds(base, B)],
                              dstb[ph].at[pl.ds(0, B)], isem[ph]).start()

    def idx_wait(ph):
        pltpu.make_async_copy(src_hbm.at[pl.ds(0, B)], idxb[ph], isem[ph]).wait()
        pltpu.make_async_copy(dst_hbm.at[pl.ds(0, B)],
                              dstb[ph].at[pl.ds(0, B)], isem[ph]).wait()

    def gather_start(ph):
        pltpu.make_async_copy(g_hbm.at[idxb[ph]], rows[ph], gsem[ph]).start()

    def gather_wait(ph):
        pltpu.make_async_copy(g_hbm.at[idxb[ph]], rows[ph], gsem[ph]).wait()

    @pl.when(nb > 0)
    def _():
        idx_start(0, 0)
        idx_wait(0)
        gather_start(0)

    @pl.when(nb > 1)
    def _():
        idx_start(1, 1)

    lanev = lax.broadcasted_iota(jnp.int32, (16,), 0)

    def accumulate(rel, ph):
        base = (b_first + rel) * B
        i_lo = jnp.maximum(e_start - base, 0)
        i_hi = jnp.minimum(e_end - base, B)
        rbuf = rows[ph]
        dbuf = dstb[ph]

        def group_body(g, prev_ld):
            off = g * 16
            dv = dbuf[pl.ds(off, 16)] - n_lo
            gi = lanev + off
            valid = (gi >= i_lo) & (gi < i_hi)
            ldv = jnp.where(valid, dv, JUNK)
            u0 = ldv[0]
            same_seg = (u0 == ldv[15]) & (u0 != JUNK) & (u0 == prev_ld)

            @pl.when(same_seg)
            def _():
                for j in range(LANES):
                    sl = pl.ds(j * 16, 16)
                    v = [rbuf[off + k, sl] for k in range(16)]
                    while len(v) > 1:
                        v = [jnp.maximum(v[2 * t], v[2 * t + 1])
                             for t in range(len(v) // 2)]
                    m_v[sl] = jnp.maximum(m_v[sl], v[0])

            @pl.when(jnp.logical_not(same_seg))
            def _():
                m = [m_v[pl.ds(j * 16, 16)] for j in range(LANES)]
                prev = prev_ld
                for k in range(16):
                    ld = ldv[k]
                    rowk = [rbuf[off + k, pl.ds(j * 16, 16)]
                            for j in range(LANES)]
                    is_new = ld != prev

                    @pl.when(is_new)
                    def _(m=m, prev=prev):
                        for j in range(LANES):
                            sl = pl.ds(j * 16, 16)
                            acc_v[prev, sl] = jnp.maximum(
                                acc_v[prev, sl], m[j])

                    m = [jnp.where(is_new, rowk[j], jnp.maximum(m[j], rowk[j]))
                         for j in range(LANES)]
                    prev = ld
                for j in range(LANES):
                    m_v[pl.ds(j * 16, 16)] = m[j]

            return ldv[15]

        prev0 = p_v[pl.ds(0, 16)][0]
        prev_out = lax.fori_loop(0, B // 16, group_body, prev0)
        p_v[pl.ds(0, 16)] = jnp.broadcast_to(prev_out, (16,))

    def pair_body(p, carry):
        for ph in range(2):
            rel = 2 * p + ph

            @pl.when(rel < nb)
            def _(rel=rel, ph=ph):
                gather_wait(ph)

                @pl.when(rel + 1 < nb)
                def _():
                    idx_wait(1 - ph)
                    gather_start(1 - ph)

                accumulate(rel, ph)

                @pl.when(rel + 2 < nb)
                def _():
                    idx_start(rel + 2, ph)
        return carry

    lax.fori_loop(0, (nb + 1) // 2, pair_body, 0)

    last_ld = p_v[pl.ds(0, 16)][0]
    for j in range(LANES):
        sl = pl.ds(j * 16, 16)
        acc_v[last_ld, sl] = jnp.maximum(acc_v[last_ld, sl], m_v[sl])

    def fin_row(r, carry):
        for j in range(LANES):
            sl = pl.ds(j * 16, 16)
            acc_v[r, sl] = acc_v[r, sl] + h_v[r, sl]
        return carry

    lax.fori_loop(0, NPW, fin_row, 0)

    @pl.when(wid < NW - 1)
    def _():
        pltpu.sync_copy(acc_v.at[pl.ds(0, NPW)], out_hbm.at[pl.ds(n_lo, NPW)])

    @pl.when(wid == NW - 1)
    def _():
        pltpu.sync_copy(acc_v.at[pl.ds(0, N_LAST)],
                        out_hbm.at[pl.ds(n_lo, N_LAST)])


def kernel(pos, x, batch, neighbors, neighbors_batch, W_xi, b_xi, W_xn, b_xn):
    del batch
    src = neighbors.astype(jnp.int32)
    dst = neighbors_batch.astype(jnp.int32)

    wp = W_xn[:C]
    wx = W_xn[C:]
    bias = (b_xi + b_xn).reshape(1, D)

    bn = 1000
    g, h = pl.pallas_call(
        _mm_body,
        grid=(N // bn,),
        in_specs=[
            pl.BlockSpec((bn, D), lambda i: (i, 0)),
            pl.BlockSpec((bn, C), lambda i: (i, 0)),
            pl.BlockSpec((D, D), lambda i: (0, 0)),
            pl.BlockSpec((C, D), lambda i: (0, 0)),
            pl.BlockSpec((D, D), lambda i: (0, 0)),
            pl.BlockSpec((1, D), lambda i: (0, 0)),
        ],
        out_specs=[
            pl.BlockSpec((bn, D), lambda i: (i, 0)),
            pl.BlockSpec((bn, D), lambda i: (i, 0)),
        ],
        out_shape=[
            jax.ShapeDtypeStruct((N, D), jnp.float32),
            jax.ShapeDtypeStruct((N, D), jnp.float32),
        ],
    )(x, pos, wx, wp, W_xi, bias)

    nb2d = dst.reshape(NB, B)
    offs = pl.pallas_call(
        _offsets_body,
        grid=(1,),
        in_specs=[pl.BlockSpec((NB, B), lambda i: (0, 0))],
        out_specs=pl.BlockSpec((1, 128), lambda i: (0, 0)),
        out_shape=jax.ShapeDtypeStruct((1, 128), jnp.int32),
    )(nb2d).reshape(128)

    mesh = plsc.VectorSubcoreMesh(core_axis_name="c", subcore_axis_name="s",
                                  num_cores=NC, num_subcores=NS)
    out = pl.kernel(
        _segmax_body,
        out_type=jax.ShapeDtypeStruct((N, D), jnp.float32),
        mesh=mesh,
        scratch_types=[
            pltpu.VMEM((NPW + 8, D), jnp.float32),
            pltpu.VMEM((NPW, D), jnp.float32),
            pltpu.VMEM((B, D), jnp.float32),
            pltpu.VMEM((B, D), jnp.float32),
            pltpu.VMEM((B,), jnp.int32),
            pltpu.VMEM((B,), jnp.int32),
            pltpu.VMEM((B + 16,), jnp.int32),
            pltpu.VMEM((B + 16,), jnp.int32),
            pltpu.VMEM((128,), jnp.int32),
            pltpu.VMEM((D,), jnp.float32),
            pltpu.VMEM((16,), jnp.int32),
            pltpu.SemaphoreType.DMA,
            pltpu.SemaphoreType.DMA,
            pltpu.SemaphoreType.DMA,
            pltpu.SemaphoreType.DMA,
        ],
    )(g, h, src, dst, offs)

    return out
